# Initial kernel scaffold; baseline (speedup 1.0000x reference)
import jax
import jax.numpy as jnp
from jax.experimental import pallas as pl


def kernel(x, edge_index, W1, b1, W2, b2, W3, b3):
    raise NotImplementedError("write your pallas kernel here")


# speedup vs baseline: 18.2834x; 18.2834x over previous
import functools

import jax
import jax.numpy as jnp
from jax import lax
from jax.experimental import pallas as pl
from jax.experimental.pallas import tpu as pltpu
from jax.experimental.pallas import tpu_sc as plsc

N = 10000
E = 320000
D_IN = 128
F1 = 16
F2 = 32
C_OUT = 40
F3P = 48

NC = 2
NS = 16
NW = NC * NS

NP = 10112
ROWS_PER_TILE = NP // NS
EP = 327680
EBLK = 128
NBLK = EP // 128
BLK_PER_TILE = NBLK // NW
PAD_DST = N

_mesh = plsc.VectorSubcoreMesh(
    core_axis_name="c", subcore_axis_name="s", num_cores=NC, num_subcores=NS
)


def _zero_rows(buf, nrows, ncols16):
    z = jnp.zeros((16,), jnp.float32)

    def body(i, carry):
        for k in range(ncols16):
            buf[i, pl.ds(k * 16, 16)] = z
        return carry

    lax.fori_loop(0, nrows, body, 0, unroll=4)


def _make_prop(F):
    @functools.partial(
        pl.kernel,
        out_type=jax.ShapeDtypeStruct((NC, NP, F), jnp.float32),
        mesh=_mesh,
        compiler_params=pltpu.CompilerParams(use_tc_tiling_on_sc=False),
        scratch_types=[
            pltpu.VMEM((BLK_PER_TILE, EBLK), jnp.int32),
            pltpu.VMEM((BLK_PER_TILE, EBLK), jnp.int32),
            pltpu.VMEM((EBLK, F), jnp.float32),
            pltpu.VMEM((ROWS_PER_TILE, F), jnp.float32),
            pltpu.VMEM_SHARED((NP, F), jnp.float32),
            pltpu.SemaphoreType.DMA,
        ],
    )
    def prop(h_hbm, src_hbm, dst_hbm, out_hbm, src_v, dst_v, rows_v, zbuf, acc, sem):
        cid = lax.axis_index("c")
        sid = lax.axis_index("s")
        wid = sid * NC + cid

        _zero_rows(zbuf, ROWS_PER_TILE, F // 16)
        row0 = sid * ROWS_PER_TILE
        pltpu.sync_copy(zbuf, acc.at[pl.ds(row0, ROWS_PER_TILE)])
        plsc.subcore_barrier()

        base = wid * BLK_PER_TILE
        pltpu.sync_copy(src_hbm.at[pl.ds(base, BLK_PER_TILE)], src_v)
        pltpu.sync_copy(dst_hbm.at[pl.ds(base, BLK_PER_TILE)], dst_v)

        def step(j, carry):
            pltpu.async_copy(h_hbm.at[src_v.at[j]], rows_v, sem).wait()
            pltpu.sync_copy(rows_v, acc.at[dst_v.at[j]], add=True)
            return carry

        lax.fori_loop(0, BLK_PER_TILE, step, 0)
        plsc.subcore_barrier()

        pltpu.sync_copy(
            acc.at[pl.ds(row0, ROWS_PER_TILE)],
            out_hbm.at[cid, pl.ds(row0, ROWS_PER_TILE)],
        )

    return prop


@functools.partial(
    pl.kernel,
    out_type=jax.ShapeDtypeStruct((NC, NP, 16), jnp.float32),
    mesh=_mesh,
    compiler_params=pltpu.CompilerParams(use_tc_tiling_on_sc=False),
    scratch_types=[
        pltpu.VMEM((BLK_PER_TILE, EBLK), jnp.int32),
        pltpu.VMEM((EBLK, 16), jnp.float32),
        pltpu.VMEM((ROWS_PER_TILE, 16), jnp.float32),
        pltpu.VMEM_SHARED((NP, 16), jnp.float32),
    ],
)
def _deg_kernel(dst_hbm, out_hbm, dst_v, ones_v, zbuf, acc):
    cid = lax.axis_index("c")
    sid = lax.axis_index("s")
    wid = sid * NC + cid

    one = jnp.ones((16,), jnp.float32)

    def fill(i, carry):
        ones_v[i, pl.ds(0, 16)] = one
        return carry

    lax.fori_loop(0, EBLK, fill, 0, unroll=4)

    _zero_rows(zbuf, ROWS_PER_TILE, 1)
    row0 = sid * ROWS_PER_TILE
    pltpu.sync_copy(zbuf, acc.at[pl.ds(row0, ROWS_PER_TILE)])
    plsc.subcore_barrier()

    base = wid * BLK_PER_TILE
    pltpu.sync_copy(dst_hbm.at[pl.ds(base, BLK_PER_TILE)], dst_v)

    def step(j, carry):
        pltpu.sync_copy(ones_v, acc.at[dst_v.at[j]], add=True)
        return carry

    lax.fori_loop(0, BLK_PER_TILE, step, 0)
    plsc.subcore_barrier()

    pltpu.sync_copy(
        acc.at[pl.ds(row0, ROWS_PER_TILE)],
        out_hbm.at[cid, pl.ds(row0, ROWS_PER_TILE)],
    )


_prop16 = _make_prop(F1)
_prop32 = _make_prop(F2)
_prop48 = _make_prop(F3P)


def _tc1_body(x_ref, w_ref, degp_ref, h_ref, dinv_ref):
    deg = degp_ref[0, :, 0] + degp_ref[1, :, 0] + 1.0
    dinv = lax.rsqrt(deg)
    dinv_ref[...] = dinv
    h = lax.dot_general(
        x_ref[...], w_ref[...], (((1,), (0,)), ((), ())),
        preferred_element_type=jnp.float32,
    )
    h_ref[...] = h * dinv[:, None]


def _tc_mid_body(sp_ref, hs_ref, dinv_ref, b_ref, w_ref, out_ref):
    dinv = dinv_ref[...]
    s = sp_ref[0] + sp_ref[1] + hs_ref[...]
    h = jnp.maximum(dinv[:, None] * s + b_ref[...], 0.0)
    h = lax.dot_general(
        h, w_ref[...], (((1,), (0,)), ((), ())),
        preferred_element_type=jnp.float32,
    )
    out_ref[...] = h * dinv[:, None]


def _tc4_body(sp_ref, hs_ref, dinv_ref, b_ref, out_ref):
    dinv = dinv_ref[...]
    h = dinv[:, None] * (sp_ref[0] + sp_ref[1] + hs_ref[...]) + b_ref[...]
    col = lax.broadcasted_iota(jnp.int32, h.shape, 1)
    h = jnp.where(col < C_OUT, h, -1e30)
    m = jnp.max(h, axis=1, keepdims=True)
    e = jnp.exp(h - m)
    out_ref[...] = (h - m) - jnp.log(jnp.sum(e, axis=1, keepdims=True))


def _tc1(x_pad, W1, degp):
    return pl.pallas_call(
        _tc1_body,
        out_shape=[
            jax.ShapeDtypeStruct((NP, F1), jnp.float32),
            jax.ShapeDtypeStruct((NP,), jnp.float32),
        ],
    )(x_pad, W1, degp)


def _tc_mid(sp, hs, dinv, b2d, W, fout):
    return pl.pallas_call(
        _tc_mid_body,
        out_shape=jax.ShapeDtypeStruct((NP, fout), jnp.float32),
    )(sp, hs, dinv, b2d, W)


def _tc4(sp, hs, dinv, b2d):
    return pl.pallas_call(
        _tc4_body,
        out_shape=jax.ShapeDtypeStruct((NP, F3P), jnp.float32),
    )(sp, hs, dinv, b2d)


def kernel(x, edge_index, W1, b1, W2, b2, W3, b3):
    x_pad = jnp.pad(x, ((0, NP - N), (0, 0)))
    pad = jnp.full((EP - E,), PAD_DST, dtype=jnp.int32)
    src2 = jnp.concatenate([edge_index[0], pad]).reshape(NBLK, EBLK)
    dst2 = jnp.concatenate([edge_index[1], pad]).reshape(NBLK, EBLK)
    W3p = jnp.pad(W3, ((0, 0), (0, F3P - C_OUT)))
    b3p = jnp.pad(b3, (0, F3P - C_OUT)).reshape(1, F3P)

    degp = _deg_kernel(dst2)
    h1s, dinv = _tc1(x_pad, W1, degp)
    s1 = _prop16(h1s, src2, dst2)
    h2s = _tc_mid(s1, h1s, dinv, b1.reshape(1, F1), W2, F2)
    s2 = _prop32(h2s, src2, dst2)
    h3s = _tc_mid(s2, h2s, dinv, b2.reshape(1, F2), W3p, F3P)
    s3 = _prop48(h3s, src2, dst2)
    out = _tc4(s3, h3s, dinv, b3p)
    return out[:N, :C_OUT]


# speedup vs baseline: 19.7485x; 1.0801x over previous
import functools

import jax
import jax.numpy as jnp
from jax import lax
from jax.experimental import pallas as pl
from jax.experimental.pallas import tpu as pltpu
from jax.experimental.pallas import tpu_sc as plsc

N = 10000
E = 320000
D_IN = 128
F1 = 16
F2 = 32
C_OUT = 40
F3P = 48

NC = 2
NS = 16
NW = NC * NS

NP = 10112
ROWS_PER_TILE = NP // NS
EP = 327680
EBLK = 128
NBLK = EP // 128
BLK_PER_TILE = NBLK // NW
PAD_DST = N

_mesh = plsc.VectorSubcoreMesh(
    core_axis_name="c", subcore_axis_name="s", num_cores=NC, num_subcores=NS
)


def _zero_rows(buf, nrows, ncols16):
    z = jnp.zeros((16,), jnp.float32)

    def body(i, carry):
        for k in range(ncols16):
            buf[i, pl.ds(k * 16, 16)] = z
        return carry

    lax.fori_loop(0, nrows, body, 0, unroll=4)


def _make_prop(F):
    @functools.partial(
        pl.kernel,
        out_type=jax.ShapeDtypeStruct((NC, NP, F), jnp.float32),
        mesh=_mesh,
        compiler_params=pltpu.CompilerParams(use_tc_tiling_on_sc=False),
        scratch_types=[
            pltpu.VMEM((BLK_PER_TILE, EBLK), jnp.int32),
            pltpu.VMEM((BLK_PER_TILE, EBLK), jnp.int32),
            pltpu.VMEM((EBLK, F), jnp.float32),
            pltpu.VMEM((EBLK, F), jnp.float32),
            pltpu.VMEM((ROWS_PER_TILE, F), jnp.float32),
            pltpu.VMEM_SHARED((NP, F), jnp.float32),
            pltpu.SemaphoreType.DMA,
        ],
    )
    def prop(h_hbm, src_hbm, dst_hbm, out_hbm, src_v, dst_v, rows_a, rows_b, zbuf, acc, sem):
        cid = lax.axis_index("c")
        sid = lax.axis_index("s")
        wid = sid * NC + cid

        _zero_rows(zbuf, ROWS_PER_TILE, F // 16)
        row0 = sid * ROWS_PER_TILE
        pltpu.sync_copy(zbuf, acc.at[pl.ds(row0, ROWS_PER_TILE)])
        plsc.subcore_barrier()

        base = wid * BLK_PER_TILE
        pltpu.sync_copy(src_hbm.at[pl.ds(base, BLK_PER_TILE)], src_v)
        pltpu.sync_copy(dst_hbm.at[pl.ds(base, BLK_PER_TILE)], dst_v)

        K = BLK_PER_TILE // 2
        pltpu.async_copy(h_hbm.at[src_v.at[0]], rows_a, sem)

        def step(k, carry):
            j0 = 2 * k
            j1 = j0 + 1
            pltpu.make_async_copy(h_hbm.at[src_v.at[j0]], rows_a, sem).wait()
            pltpu.async_copy(h_hbm.at[src_v.at[j1]], rows_b, sem)
            pltpu.sync_copy(rows_a, acc.at[dst_v.at[j0]], add=True)
            pltpu.make_async_copy(h_hbm.at[src_v.at[j1]], rows_b, sem).wait()

            @pl.when(k < K - 1)
            def _():
                pltpu.async_copy(h_hbm.at[src_v.at[j1 + 1]], rows_a, sem)

            pltpu.sync_copy(rows_b, acc.at[dst_v.at[j1]], add=True)
            return carry

        lax.fori_loop(0, K, step, 0)
        plsc.subcore_barrier()

        pltpu.sync_copy(
            acc.at[pl.ds(row0, ROWS_PER_TILE)],
            out_hbm.at[cid, pl.ds(row0, ROWS_PER_TILE)],
        )

    return prop


@functools.partial(
    pl.kernel,
    out_type=jax.ShapeDtypeStruct((NC, NP, 16), jnp.float32),
    mesh=_mesh,
    compiler_params=pltpu.CompilerParams(use_tc_tiling_on_sc=False),
    scratch_types=[
        pltpu.VMEM((BLK_PER_TILE, EBLK), jnp.int32),
        pltpu.VMEM((EBLK, 16), jnp.float32),
        pltpu.VMEM((ROWS_PER_TILE, 16), jnp.float32),
        pltpu.VMEM_SHARED((NP, 16), jnp.float32),
    ],
)
def _deg_kernel(dst_hbm, out_hbm, dst_v, ones_v, zbuf, acc):
    cid = lax.axis_index("c")
    sid = lax.axis_index("s")
    wid = sid * NC + cid

    one = jnp.ones((16,), jnp.float32)

    def fill(i, carry):
        ones_v[i, pl.ds(0, 16)] = one
        return carry

    lax.fori_loop(0, EBLK, fill, 0, unroll=4)

    _zero_rows(zbuf, ROWS_PER_TILE, 1)
    row0 = sid * ROWS_PER_TILE
    pltpu.sync_copy(zbuf, acc.at[pl.ds(row0, ROWS_PER_TILE)])
    plsc.subcore_barrier()

    base = wid * BLK_PER_TILE
    pltpu.sync_copy(dst_hbm.at[pl.ds(base, BLK_PER_TILE)], dst_v)

    def step(j, carry):
        pltpu.sync_copy(ones_v, acc.at[dst_v.at[j]], add=True)
        return carry

    lax.fori_loop(0, BLK_PER_TILE, step, 0)
    plsc.subcore_barrier()

    pltpu.sync_copy(
        acc.at[pl.ds(row0, ROWS_PER_TILE)],
        out_hbm.at[cid, pl.ds(row0, ROWS_PER_TILE)],
    )


_prop16 = _make_prop(F1)
_prop32 = _make_prop(F2)
_prop48 = _make_prop(F3P)


def _tc1_body(x_ref, w_ref, degp_ref, h_ref, dinv_ref):
    deg = degp_ref[0, :, 0] + degp_ref[1, :, 0] + 1.0
    dinv = lax.rsqrt(deg)
    dinv_ref[...] = dinv
    h = lax.dot_general(
        x_ref[...], w_ref[...], (((1,), (0,)), ((), ())),
        preferred_element_type=jnp.float32,
    )
    h_ref[...] = h * dinv[:, None]


def _tc_mid_body(sp_ref, hs_ref, dinv_ref, b_ref, w_ref, out_ref):
    dinv = dinv_ref[...]
    s = sp_ref[0] + sp_ref[1] + hs_ref[...]
    h = jnp.maximum(dinv[:, None] * s + b_ref[...], 0.0)
    h = lax.dot_general(
        h, w_ref[...], (((1,), (0,)), ((), ())),
        preferred_element_type=jnp.float32,
    )
    out_ref[...] = h * dinv[:, None]


def _tc4_body(sp_ref, hs_ref, dinv_ref, b_ref, out_ref):
    dinv = dinv_ref[...]
    h = dinv[:, None] * (sp_ref[0] + sp_ref[1] + hs_ref[...]) + b_ref[...]
    col = lax.broadcasted_iota(jnp.int32, h.shape, 1)
    h = jnp.where(col < C_OUT, h, -1e30)
    m = jnp.max(h, axis=1, keepdims=True)
    e = jnp.exp(h - m)
    out_ref[...] = (h - m) - jnp.log(jnp.sum(e, axis=1, keepdims=True))


def _tc1(x_pad, W1, degp):
    return pl.pallas_call(
        _tc1_body,
        out_shape=[
            jax.ShapeDtypeStruct((NP, F1), jnp.float32),
            jax.ShapeDtypeStruct((NP,), jnp.float32),
        ],
    )(x_pad, W1, degp)


def _tc_mid(sp, hs, dinv, b2d, W, fout):
    return pl.pallas_call(
        _tc_mid_body,
        out_shape=jax.ShapeDtypeStruct((NP, fout), jnp.float32),
    )(sp, hs, dinv, b2d, W)


def _tc4(sp, hs, dinv, b2d):
    return pl.pallas_call(
        _tc4_body,
        out_shape=jax.ShapeDtypeStruct((NP, F3P), jnp.float32),
    )(sp, hs, dinv, b2d)


def kernel(x, edge_index, W1, b1, W2, b2, W3, b3):
    x_pad = jnp.pad(x, ((0, NP - N), (0, 0)))
    pad = jnp.full((EP - E,), PAD_DST, dtype=jnp.int32)
    src2 = jnp.concatenate([edge_index[0], pad]).reshape(NBLK, EBLK)
    dst2 = jnp.concatenate([edge_index[1], pad]).reshape(NBLK, EBLK)
    W3p = jnp.pad(W3, ((0, 0), (0, F3P - C_OUT)))
    b3p = jnp.pad(b3, (0, F3P - C_OUT)).reshape(1, F3P)

    degp = _deg_kernel(dst2)
    h1s, dinv = _tc1(x_pad, W1, degp)
    s1 = _prop16(h1s, src2, dst2)
    h2s = _tc_mid(s1, h1s, dinv, b1.reshape(1, F1), W2, F2)
    s2 = _prop32(h2s, src2, dst2)
    h3s = _tc_mid(s2, h2s, dinv, b2.reshape(1, F2), W3p, F3P)
    s3 = _prop48(h3s, src2, dst2)
    out = _tc4(s3, h3s, dinv, b3p)
    return out[:N, :C_OUT]


# speedup vs baseline: 22.4143x; 1.1350x over previous
import functools

import jax
import jax.numpy as jnp
from jax import lax
from jax.experimental import pallas as pl
from jax.experimental.pallas import tpu as pltpu
from jax.experimental.pallas import tpu_sc as plsc

N = 10000
E = 320000
D_IN = 128
F1 = 16
F2 = 32
C_OUT = 40
F3P = 48

NC = 2
NS = 16
NW = NC * NS

NP = 10112
ROWS_PER_TILE = NP // NS
EP = 327680
EBLK = 128
NBLK = EP // 128
BLK_PER_TILE = NBLK // NW
PAD_DST = N
GK = 4

_mesh = plsc.VectorSubcoreMesh(
    core_axis_name="c", subcore_axis_name="s", num_cores=NC, num_subcores=NS
)


def _zero_rows(buf, nrows, ncols16):
    z = jnp.zeros((16,), jnp.float32)

    def body(i, carry):
        for k in range(ncols16):
            buf[i, pl.ds(k * 16, 16)] = z
        return carry

    lax.fori_loop(0, nrows, body, 0, unroll=4)


def _make_prop(F):
    @functools.partial(
        pl.kernel,
        out_type=jax.ShapeDtypeStruct((NC, NP, F), jnp.float32),
        mesh=_mesh,
        compiler_params=pltpu.CompilerParams(use_tc_tiling_on_sc=False),
        scratch_types=[
            pltpu.VMEM((BLK_PER_TILE, EBLK), jnp.int32),
            pltpu.VMEM((BLK_PER_TILE, EBLK), jnp.int32),
            pltpu.VMEM((2 * GK * EBLK, F), jnp.float32),
            pltpu.VMEM((ROWS_PER_TILE, F), jnp.float32),
            pltpu.VMEM_SHARED((NP, F), jnp.float32),
            pltpu.SemaphoreType.DMA,
            pltpu.SemaphoreType.DMA,
        ],
    )
    def prop(h_hbm, src_hbm, dst_hbm, out_hbm, src_v, dst_v, rows_v, zbuf, acc, gsem, ssem):
        cid = lax.axis_index("c")
        sid = lax.axis_index("s")
        wid = sid * NC + cid

        _zero_rows(zbuf, ROWS_PER_TILE, F // 16)
        row0 = sid * ROWS_PER_TILE
        pltpu.sync_copy(zbuf, acc.at[pl.ds(row0, ROWS_PER_TILE)])
        plsc.subcore_barrier()

        base = wid * BLK_PER_TILE
        pltpu.sync_copy(src_hbm.at[pl.ds(base, BLK_PER_TILE)], src_v)
        pltpu.sync_copy(dst_hbm.at[pl.ds(base, BLK_PER_TILE)], dst_v)

        G = BLK_PER_TILE // GK

        def buf(g, b):
            return rows_v.at[pl.ds((g * GK + b) * EBLK, EBLK)]

        def fire_gathers(s, g):
            for b in range(GK):
                j = jnp.minimum(s * GK + b, BLK_PER_TILE - 1)
                pltpu.async_copy(h_hbm.at[src_v.at[j]], buf(g, b), gsem)

        def drain_gathers(s, g):
            for b in range(GK):
                j = s * GK + b
                pltpu.make_async_copy(h_hbm.at[src_v.at[j]], buf(g, b), gsem).wait()

        def fire_scatters(s, g):
            for b in range(GK):
                j = s * GK + b
                pltpu.async_copy(buf(g, b), acc.at[dst_v.at[j]], ssem, add=True)

        def drain_scatters(s, g):
            for b in range(GK):
                j = s * GK + b
                pltpu.make_async_copy(buf(g, b), acc.at[dst_v.at[j]], ssem).wait()

        fire_gathers(jnp.int32(0), jnp.int32(0))

        def step(s, carry):
            g = lax.rem(s, 2)

            drain_gathers(s, g)

            @pl.when(s > 0)
            def _():
                drain_scatters(s - 1, 1 - g)

            @pl.when(s < G - 1)
            def _():
                fire_gathers(s + 1, 1 - g)

            fire_scatters(s, g)
            return carry

        lax.fori_loop(0, G, step, 0)
        drain_scatters(jnp.int32(G - 1), jnp.int32((G - 1) % 2))
        plsc.subcore_barrier()

        pltpu.sync_copy(
            acc.at[pl.ds(row0, ROWS_PER_TILE)],
            out_hbm.at[cid, pl.ds(row0, ROWS_PER_TILE)],
        )

    return prop


@functools.partial(
    pl.kernel,
    out_type=jax.ShapeDtypeStruct((NC, NP, 16), jnp.float32),
    mesh=_mesh,
    compiler_params=pltpu.CompilerParams(use_tc_tiling_on_sc=False),
    scratch_types=[
        pltpu.VMEM((BLK_PER_TILE, EBLK), jnp.int32),
        pltpu.VMEM((EBLK, 16), jnp.float32),
        pltpu.VMEM((ROWS_PER_TILE, 16), jnp.float32),
        pltpu.VMEM_SHARED((NP, 16), jnp.float32),
    ],
)
def _deg_kernel(dst_hbm, out_hbm, dst_v, ones_v, zbuf, acc):
    cid = lax.axis_index("c")
    sid = lax.axis_index("s")
    wid = sid * NC + cid

    one = jnp.ones((16,), jnp.float32)

    def fill(i, carry):
        ones_v[i, pl.ds(0, 16)] = one
        return carry

    lax.fori_loop(0, EBLK, fill, 0, unroll=4)

    _zero_rows(zbuf, ROWS_PER_TILE, 1)
    row0 = sid * ROWS_PER_TILE
    pltpu.sync_copy(zbuf, acc.at[pl.ds(row0, ROWS_PER_TILE)])
    plsc.subcore_barrier()

    base = wid * BLK_PER_TILE
    pltpu.sync_copy(dst_hbm.at[pl.ds(base, BLK_PER_TILE)], dst_v)

    def step(j, carry):
        pltpu.sync_copy(ones_v, acc.at[dst_v.at[j]], add=True)
        return carry

    lax.fori_loop(0, BLK_PER_TILE, step, 0)
    plsc.subcore_barrier()

    pltpu.sync_copy(
        acc.at[pl.ds(row0, ROWS_PER_TILE)],
        out_hbm.at[cid, pl.ds(row0, ROWS_PER_TILE)],
    )


_prop16 = _make_prop(F1)
_prop32 = _make_prop(F2)
_prop48 = _make_prop(F3P)


def _tc1_body(x_ref, w_ref, degp_ref, h_ref, dinv_ref):
    deg = degp_ref[0, :, 0] + degp_ref[1, :, 0] + 1.0
    dinv = lax.rsqrt(deg)
    dinv_ref[...] = dinv
    h = lax.dot_general(
        x_ref[...], w_ref[...], (((1,), (0,)), ((), ())),
        preferred_element_type=jnp.float32,
    )
    h_ref[...] = h * dinv[:, None]


def _tc_mid_body(sp_ref, hs_ref, dinv_ref, b_ref, w_ref, out_ref):
    dinv = dinv_ref[...]
    s = sp_ref[0] + sp_ref[1] + hs_ref[...]
    h = jnp.maximum(dinv[:, None] * s + b_ref[...], 0.0)
    h = lax.dot_general(
        h, w_ref[...], (((1,), (0,)), ((), ())),
        preferred_element_type=jnp.float32,
    )
    out_ref[...] = h * dinv[:, None]


def _tc4_body(sp_ref, hs_ref, dinv_ref, b_ref, out_ref):
    dinv = dinv_ref[...]
    h = dinv[:, None] * (sp_ref[0] + sp_ref[1] + hs_ref[...]) + b_ref[...]
    col = lax.broadcasted_iota(jnp.int32, h.shape, 1)
    h = jnp.where(col < C_OUT, h, -1e30)
    m = jnp.max(h, axis=1, keepdims=True)
    e = jnp.exp(h - m)
    out_ref[...] = (h - m) - jnp.log(jnp.sum(e, axis=1, keepdims=True))


def _tc1(x_pad, W1, degp):
    return pl.pallas_call(
        _tc1_body,
        out_shape=[
            jax.ShapeDtypeStruct((NP, F1), jnp.float32),
            jax.ShapeDtypeStruct((NP,), jnp.float32),
        ],
    )(x_pad, W1, degp)


def _tc_mid(sp, hs, dinv, b2d, W, fout):
    return pl.pallas_call(
        _tc_mid_body,
        out_shape=jax.ShapeDtypeStruct((NP, fout), jnp.float32),
    )(sp, hs, dinv, b2d, W)


def _tc4(sp, hs, dinv, b2d):
    return pl.pallas_call(
        _tc4_body,
        out_shape=jax.ShapeDtypeStruct((NP, F3P), jnp.float32),
    )(sp, hs, dinv, b2d)


def kernel(x, edge_index, W1, b1, W2, b2, W3, b3):
    x_pad = jnp.pad(x, ((0, NP - N), (0, 0)))
    pad = jnp.full((EP - E,), PAD_DST, dtype=jnp.int32)
    src2 = jnp.concatenate([edge_index[0], pad]).reshape(NBLK, EBLK)
    dst2 = jnp.concatenate([edge_index[1], pad]).reshape(NBLK, EBLK)
    W3p = jnp.pad(W3, ((0, 0), (0, F3P - C_OUT)))
    b3p = jnp.pad(b3, (0, F3P - C_OUT)).reshape(1, F3P)

    degp = _deg_kernel(dst2)
    h1s, dinv = _tc1(x_pad, W1, degp)
    s1 = _prop16(h1s, src2, dst2)
    h2s = _tc_mid(s1, h1s, dinv, b1.reshape(1, F1), W2, F2)
    s2 = _prop32(h2s, src2, dst2)
    h3s = _tc_mid(s2, h2s, dinv, b2.reshape(1, F2), W3p, F3P)
    s3 = _prop48(h3s, src2, dst2)
    out = _tc4(s3, h3s, dinv, b3p)
    return out[:N, :C_OUT]


# speedup vs baseline: 47.3049x; 2.1105x over previous
import functools

import jax
import jax.numpy as jnp
from jax import lax
from jax.experimental import pallas as pl
from jax.experimental.pallas import tpu as pltpu
from jax.experimental.pallas import tpu_sc as plsc

N = 10000
E = 320000
D_IN = 128
F1 = 16
F2 = 32
C_OUT = 40
F3P = 48

NC = 2
NS = 16
NW = NC * NS

NP = 10112
ROWS_PER_TILE = NP // NS
EP = 327680
EBLK = 512
EDGES_PER_TILE = EP // NW
BLK_PER_TILE = EDGES_PER_TILE // EBLK
NPAD_ROWS = NP - N
ZROWS = 160

_mesh = plsc.VectorSubcoreMesh(
    core_axis_name="c", subcore_axis_name="s", num_cores=NC, num_subcores=NS
)


def _zero_rows(buf, nrows, ncols16):
    z = jnp.zeros((16,), jnp.float32)

    def body(i, carry):
        for k in range(ncols16):
            buf[i, pl.ds(k * 16, 16)] = z
        return carry

    lax.fori_loop(0, nrows, body, 0, unroll=4)


def _make_prop(F, GK):
    @functools.partial(
        pl.kernel,
        out_type=jax.ShapeDtypeStruct((NC, NP, F), jnp.float32),
        mesh=_mesh,
        compiler_params=pltpu.CompilerParams(use_tc_tiling_on_sc=False),
        scratch_types=[
            pltpu.VMEM((EDGES_PER_TILE,), jnp.int32),
            pltpu.VMEM((EDGES_PER_TILE,), jnp.int32),
            pltpu.VMEM((2 * GK * EBLK, F), jnp.float32),
            pltpu.VMEM((ZROWS, F), jnp.float32),
            pltpu.VMEM_SHARED((NP, F), jnp.float32),
            pltpu.SemaphoreType.DMA,
            pltpu.SemaphoreType.DMA,
        ],
    )
    def prop(h_hbm, src_hbm, dst_hbm, out_hbm, src_v, dst_v, rows_v, zbuf, acc, gsem, ssem):
        cid = lax.axis_index("c")
        sid = lax.axis_index("s")
        wid = sid * NC + cid

        _zero_rows(zbuf, ZROWS, F // 16)
        row0 = sid * ROWS_PER_TILE
        off = 0
        while off < ROWS_PER_TILE:
            n = min(ZROWS, ROWS_PER_TILE - off)
            pltpu.sync_copy(zbuf.at[pl.ds(0, n)], acc.at[pl.ds(row0 + off, n)])
            off += n
        plsc.subcore_barrier()

        pltpu.sync_copy(src_hbm.at[wid], src_v)
        pltpu.sync_copy(dst_hbm.at[wid], dst_v)

        G = BLK_PER_TILE // GK

        def buf(g, b):
            return rows_v.at[pl.ds((g * GK + b) * EBLK, EBLK)]

        def idx(v, j):
            return v.at[pl.ds(j * EBLK, EBLK)]

        def fire_gathers(s, g):
            for b in range(GK):
                j = jnp.minimum(s * GK + b, BLK_PER_TILE - 1)
                pltpu.async_copy(h_hbm.at[idx(src_v, j)], buf(g, b), gsem)

        def drain_gathers(s, g):
            for b in range(GK):
                j = s * GK + b
                pltpu.make_async_copy(h_hbm.at[idx(src_v, j)], buf(g, b), gsem).wait()

        def fire_scatters(s, g):
            for b in range(GK):
                j = s * GK + b
                pltpu.async_copy(buf(g, b), acc.at[idx(dst_v, j)], ssem, add=True)

        def drain_scatters(s, g):
            for b in range(GK):
                j = s * GK + b
                pltpu.make_async_copy(buf(g, b), acc.at[idx(dst_v, j)], ssem).wait()

        fire_gathers(jnp.int32(0), jnp.int32(0))

        def step(s, carry):
            g = lax.rem(s, 2)

            drain_gathers(s, g)

            @pl.when(s > 0)
            def _():
                drain_scatters(s - 1, 1 - g)

            @pl.when(s < G - 1)
            def _():
                fire_gathers(s + 1, 1 - g)

            fire_scatters(s, g)
            return carry

        lax.fori_loop(0, G, step, 0)
        drain_scatters(jnp.int32(G - 1), jnp.int32((G - 1) % 2))
        plsc.subcore_barrier()

        pltpu.sync_copy(
            acc.at[pl.ds(row0, ROWS_PER_TILE)],
            out_hbm.at[cid, pl.ds(row0, ROWS_PER_TILE)],
        )

    return prop


@functools.partial(
    pl.kernel,
    out_type=jax.ShapeDtypeStruct((NC, NP, 16), jnp.float32),
    mesh=_mesh,
    compiler_params=pltpu.CompilerParams(use_tc_tiling_on_sc=False),
    scratch_types=[
        pltpu.VMEM((EDGES_PER_TILE,), jnp.int32),
        pltpu.VMEM((EBLK, 16), jnp.float32),
        pltpu.VMEM((ROWS_PER_TILE, 16), jnp.float32),
        pltpu.VMEM_SHARED((NP, 16), jnp.float32),
    ],
)
def _deg_kernel(dst_hbm, out_hbm, dst_v, ones_v, zbuf, acc):
    cid = lax.axis_index("c")
    sid = lax.axis_index("s")
    wid = sid * NC + cid

    one = jnp.ones((16,), jnp.float32)

    def fill(i, carry):
        ones_v[i, pl.ds(0, 16)] = one
        return carry

    lax.fori_loop(0, EBLK, fill, 0, unroll=4)

    _zero_rows(zbuf, ROWS_PER_TILE, 1)
    row0 = sid * ROWS_PER_TILE
    pltpu.sync_copy(zbuf, acc.at[pl.ds(row0, ROWS_PER_TILE)])
    plsc.subcore_barrier()

    pltpu.sync_copy(dst_hbm.at[wid], dst_v)

    def step(j, carry):
        pltpu.sync_copy(ones_v, acc.at[dst_v.at[pl.ds(j * EBLK, EBLK)]], add=True)
        return carry

    lax.fori_loop(0, BLK_PER_TILE, step, 0)
    plsc.subcore_barrier()

    pltpu.sync_copy(
        acc.at[pl.ds(row0, ROWS_PER_TILE)],
        out_hbm.at[cid, pl.ds(row0, ROWS_PER_TILE)],
    )


_prop16 = _make_prop(F1, 4)
_prop32 = _make_prop(F2, 2)
_prop48 = _make_prop(F3P, 1)


def _tc1_body(x_ref, w_ref, degp_ref, h_ref, dinv_ref):
    deg = degp_ref[0, :, 0] + degp_ref[1, :, 0] + 1.0
    dinv = lax.rsqrt(deg)
    dinv_ref[...] = dinv
    h = lax.dot_general(
        x_ref[...], w_ref[...], (((1,), (0,)), ((), ())),
        preferred_element_type=jnp.float32,
    )
    h_ref[...] = h * dinv[:, None]


def _tc_mid_body(sp_ref, hs_ref, dinv_ref, b_ref, w_ref, out_ref):
    dinv = dinv_ref[...]
    s = sp_ref[0] + sp_ref[1] + hs_ref[...]
    h = jnp.maximum(dinv[:, None] * s + b_ref[...], 0.0)
    h = lax.dot_general(
        h, w_ref[...], (((1,), (0,)), ((), ())),
        preferred_element_type=jnp.float32,
    )
    out_ref[...] = h * dinv[:, None]


def _tc4_body(sp_ref, hs_ref, dinv_ref, b_ref, out_ref):
    dinv = dinv_ref[...]
    h = dinv[:, None] * (sp_ref[0] + sp_ref[1] + hs_ref[...]) + b_ref[...]
    col = lax.broadcasted_iota(jnp.int32, h.shape, 1)
    h = jnp.where(col < C_OUT, h, -1e30)
    m = jnp.max(h, axis=1, keepdims=True)
    e = jnp.exp(h - m)
    out_ref[...] = (h - m) - jnp.log(jnp.sum(e, axis=1, keepdims=True))


def _tc1(x_pad, W1, degp):
    return pl.pallas_call(
        _tc1_body,
        out_shape=[
            jax.ShapeDtypeStruct((NP, F1), jnp.float32),
            jax.ShapeDtypeStruct((NP,), jnp.float32),
        ],
    )(x_pad, W1, degp)


def _tc_mid(sp, hs, dinv, b2d, W, fout):
    return pl.pallas_call(
        _tc_mid_body,
        out_shape=jax.ShapeDtypeStruct((NP, fout), jnp.float32),
    )(sp, hs, dinv, b2d, W)


def _tc4(sp, hs, dinv, b2d):
    return pl.pallas_call(
        _tc4_body,
        out_shape=jax.ShapeDtypeStruct((NP, F3P), jnp.float32),
    )(sp, hs, dinv, b2d)


def kernel(x, edge_index, W1, b1, W2, b2, W3, b3):
    x_pad = jnp.pad(x, ((0, NP - N), (0, 0)))
    pad = N + jnp.arange(EP - E, dtype=jnp.int32) % NPAD_ROWS
    src2 = jnp.concatenate([edge_index[0], pad]).reshape(NW, EDGES_PER_TILE)
    dst2 = jnp.concatenate([edge_index[1], pad]).reshape(NW, EDGES_PER_TILE)
    W3p = jnp.pad(W3, ((0, 0), (0, F3P - C_OUT)))
    b3p = jnp.pad(b3, (0, F3P - C_OUT)).reshape(1, F3P)

    degp = _deg_kernel(dst2)
    h1s, dinv = _tc1(x_pad, W1, degp)
    s1 = _prop16(h1s, src2, dst2)
    h2s = _tc_mid(s1, h1s, dinv, b1.reshape(1, F1), W2, F2)
    s2 = _prop32(h2s, src2, dst2)
    h3s = _tc_mid(s2, h2s, dinv, b2.reshape(1, F2), W3p, F3P)
    s3 = _prop48(h3s, src2, dst2)
    out = _tc4(s3, h3s, dinv, b3p)
    return out[:N, :C_OUT]


# speedup vs baseline: 47.3479x; 1.0009x over previous
import functools

import jax
import jax.numpy as jnp
from jax import lax
from jax.experimental import pallas as pl
from jax.experimental.pallas import tpu as pltpu
from jax.experimental.pallas import tpu_sc as plsc

N = 10000
E = 320000
D_IN = 128
F1 = 16
F2 = 32
C_OUT = 40
F3P = 48

NC = 2
NS = 16
NW = NC * NS

NP = 10112
ROWS_PER_TILE = NP // NS
EP = 327680
EBLK = 512
EDGES_PER_TILE = EP // NW
BLK_PER_TILE = EDGES_PER_TILE // EBLK
NPAD_ROWS = NP - N
ZROWS = 160

_mesh = plsc.VectorSubcoreMesh(
    core_axis_name="c", subcore_axis_name="s", num_cores=NC, num_subcores=NS
)


def _zero_rows(buf, nrows, ncols16):
    z = jnp.zeros((16,), jnp.float32)

    def body(i, carry):
        for k in range(ncols16):
            buf[i, pl.ds(k * 16, 16)] = z
        return carry

    lax.fori_loop(0, nrows, body, 0, unroll=4)


def _make_prop(F, GK, EBLK):
    BLK_PER_TILE = EDGES_PER_TILE // EBLK

    @functools.partial(
        pl.kernel,
        out_type=jax.ShapeDtypeStruct((NC, NP, F), jnp.float32),
        mesh=_mesh,
        compiler_params=pltpu.CompilerParams(use_tc_tiling_on_sc=False),
        scratch_types=[
            pltpu.VMEM((EDGES_PER_TILE,), jnp.int32),
            pltpu.VMEM((EDGES_PER_TILE,), jnp.int32),
            pltpu.VMEM((2 * GK * EBLK, F), jnp.float32),
            pltpu.VMEM((ZROWS, F), jnp.float32),
            pltpu.VMEM_SHARED((NP, F), jnp.float32),
            pltpu.SemaphoreType.DMA,
            pltpu.SemaphoreType.DMA,
        ],
    )
    def prop(h_hbm, src_hbm, dst_hbm, out_hbm, src_v, dst_v, rows_v, zbuf, acc, gsem, ssem):
        cid = lax.axis_index("c")
        sid = lax.axis_index("s")
        wid = sid * NC + cid

        _zero_rows(zbuf, ZROWS, F // 16)
        row0 = sid * ROWS_PER_TILE
        off = 0
        while off < ROWS_PER_TILE:
            n = min(ZROWS, ROWS_PER_TILE - off)
            pltpu.sync_copy(zbuf.at[pl.ds(0, n)], acc.at[pl.ds(row0 + off, n)])
            off += n
        plsc.subcore_barrier()

        pltpu.sync_copy(src_hbm.at[wid], src_v)
        pltpu.sync_copy(dst_hbm.at[wid], dst_v)

        G = BLK_PER_TILE // GK

        def buf(g, b):
            return rows_v.at[pl.ds((g * GK + b) * EBLK, EBLK)]

        def idx(v, j):
            return v.at[pl.ds(j * EBLK, EBLK)]

        def fire_gathers(s, g):
            for b in range(GK):
                j = jnp.minimum(s * GK + b, BLK_PER_TILE - 1)
                pltpu.async_copy(h_hbm.at[idx(src_v, j)], buf(g, b), gsem)

        def drain_gathers(s, g):
            for b in range(GK):
                j = s * GK + b
                pltpu.make_async_copy(h_hbm.at[idx(src_v, j)], buf(g, b), gsem).wait()

        def fire_scatters(s, g):
            for b in range(GK):
                j = s * GK + b
                pltpu.async_copy(buf(g, b), acc.at[idx(dst_v, j)], ssem, add=True)

        def drain_scatters(s, g):
            for b in range(GK):
                j = s * GK + b
                pltpu.make_async_copy(buf(g, b), acc.at[idx(dst_v, j)], ssem).wait()

        fire_gathers(jnp.int32(0), jnp.int32(0))

        def step(s, carry):
            g = lax.rem(s, 2)

            drain_gathers(s, g)

            @pl.when(s > 0)
            def _():
                drain_scatters(s - 1, 1 - g)

            @pl.when(s < G - 1)
            def _():
                fire_gathers(s + 1, 1 - g)

            fire_scatters(s, g)
            return carry

        lax.fori_loop(0, G, step, 0)
        drain_scatters(jnp.int32(G - 1), jnp.int32((G - 1) % 2))
        plsc.subcore_barrier()

        pltpu.sync_copy(
            acc.at[pl.ds(row0, ROWS_PER_TILE)],
            out_hbm.at[cid, pl.ds(row0, ROWS_PER_TILE)],
        )

    return prop


@functools.partial(
    pl.kernel,
    out_type=jax.ShapeDtypeStruct((NC, NP, 16), jnp.float32),
    mesh=_mesh,
    compiler_params=pltpu.CompilerParams(use_tc_tiling_on_sc=False),
    scratch_types=[
        pltpu.VMEM((EDGES_PER_TILE,), jnp.int32),
        pltpu.VMEM((EBLK, 16), jnp.float32),
        pltpu.VMEM((ROWS_PER_TILE, 16), jnp.float32),
        pltpu.VMEM_SHARED((NP, 16), jnp.float32),
    ],
)
def _deg_kernel(dst_hbm, out_hbm, dst_v, ones_v, zbuf, acc):
    cid = lax.axis_index("c")
    sid = lax.axis_index("s")
    wid = sid * NC + cid

    one = jnp.ones((16,), jnp.float32)

    def fill(i, carry):
        ones_v[i, pl.ds(0, 16)] = one
        return carry

    lax.fori_loop(0, EBLK, fill, 0, unroll=4)

    _zero_rows(zbuf, ROWS_PER_TILE, 1)
    row0 = sid * ROWS_PER_TILE
    pltpu.sync_copy(zbuf, acc.at[pl.ds(row0, ROWS_PER_TILE)])
    plsc.subcore_barrier()

    pltpu.sync_copy(dst_hbm.at[wid], dst_v)

    def step(j, carry):
        pltpu.sync_copy(ones_v, acc.at[dst_v.at[pl.ds(j * EBLK, EBLK)]], add=True)
        return carry

    lax.fori_loop(0, BLK_PER_TILE, step, 0)
    plsc.subcore_barrier()

    pltpu.sync_copy(
        acc.at[pl.ds(row0, ROWS_PER_TILE)],
        out_hbm.at[cid, pl.ds(row0, ROWS_PER_TILE)],
    )


_prop16 = _make_prop(F1, 2, 1024)
_prop32 = _make_prop(F2, 1, 1024)
_prop48 = _make_prop(F3P, 1, 512)


def _tc0_body(x_ref, w_ref, h_ref):
    h_ref[...] = lax.dot_general(
        x_ref[...], w_ref[...], (((1,), (0,)), ((), ())),
        preferred_element_type=jnp.float32,
    )


def _tc1_body(h1_ref, degp_ref, h_ref, dinv_ref):
    deg = degp_ref[0, :, 0] + degp_ref[1, :, 0] + 1.0
    dinv = lax.rsqrt(deg)
    dinv_ref[...] = dinv
    h_ref[...] = h1_ref[...] * dinv[:, None]


def _tc_mid_body(sp_ref, hs_ref, dinv_ref, b_ref, w_ref, out_ref):
    dinv = dinv_ref[...]
    s = sp_ref[0] + sp_ref[1] + hs_ref[...]
    h = jnp.maximum(dinv[:, None] * s + b_ref[...], 0.0)
    h = lax.dot_general(
        h, w_ref[...], (((1,), (0,)), ((), ())),
        preferred_element_type=jnp.float32,
    )
    out_ref[...] = h * dinv[:, None]


def _tc4_body(sp_ref, hs_ref, dinv_ref, b_ref, out_ref):
    dinv = dinv_ref[...]
    h = dinv[:, None] * (sp_ref[0] + sp_ref[1] + hs_ref[...]) + b_ref[...]
    col = lax.broadcasted_iota(jnp.int32, h.shape, 1)
    h = jnp.where(col < C_OUT, h, -1e30)
    m = jnp.max(h, axis=1, keepdims=True)
    e = jnp.exp(h - m)
    out_ref[...] = (h - m) - jnp.log(jnp.sum(e, axis=1, keepdims=True))


def _tc0(x_pad, W1):
    return pl.pallas_call(
        _tc0_body,
        out_shape=jax.ShapeDtypeStruct((NP, F1), jnp.float32),
    )(x_pad, W1)


def _tc1(h1, degp):
    return pl.pallas_call(
        _tc1_body,
        out_shape=[
            jax.ShapeDtypeStruct((NP, F1), jnp.float32),
            jax.ShapeDtypeStruct((NP,), jnp.float32),
        ],
    )(h1, degp)


def _tc_mid(sp, hs, dinv, b2d, W, fout):
    return pl.pallas_call(
        _tc_mid_body,
        out_shape=jax.ShapeDtypeStruct((NP, fout), jnp.float32),
    )(sp, hs, dinv, b2d, W)


def _tc4(sp, hs, dinv, b2d):
    return pl.pallas_call(
        _tc4_body,
        out_shape=jax.ShapeDtypeStruct((NP, F3P), jnp.float32),
    )(sp, hs, dinv, b2d)


def kernel(x, edge_index, W1, b1, W2, b2, W3, b3):
    x_pad = jnp.pad(x, ((0, NP - N), (0, 0)))
    pad = N + jnp.arange(EP - E, dtype=jnp.int32) % NPAD_ROWS
    src2 = jnp.concatenate([edge_index[0], pad]).reshape(NW, EDGES_PER_TILE)
    dst2 = jnp.concatenate([edge_index[1], pad]).reshape(NW, EDGES_PER_TILE)
    W3p = jnp.pad(W3, ((0, 0), (0, F3P - C_OUT)))
    b3p = jnp.pad(b3, (0, F3P - C_OUT)).reshape(1, F3P)

    degp = _deg_kernel(dst2)
    h1 = _tc0(x_pad, W1)
    h1s, dinv = _tc1(h1, degp)
    s1 = _prop16(h1s, src2, dst2)
    h2s = _tc_mid(s1, h1s, dinv, b1.reshape(1, F1), W2, F2)
    s2 = _prop32(h2s, src2, dst2)
    h3s = _tc_mid(s2, h2s, dinv, b2.reshape(1, F2), W3p, F3P)
    s3 = _prop48(h3s, src2, dst2)
    out = _tc4(s3, h3s, dinv, b3p)
    return out[:N, :C_OUT]


# speedup vs baseline: 50.0327x; 1.0567x over previous
import functools

import jax
import jax.numpy as jnp
from jax import lax
from jax.experimental import pallas as pl
from jax.experimental.pallas import tpu as pltpu
from jax.experimental.pallas import tpu_sc as plsc

N = 10000
E = 320000
D_IN = 128
F1 = 16
F2 = 32
C_OUT = 40
F3P = 48

NC = 2
NS = 16
NW = NC * NS

NP = 10112
ROWS_PER_TILE = NP // NS
EDGES_PER_TILE = E // NW
EBLK = 1000
BLK_PER_TILE = EDGES_PER_TILE // EBLK
ZROWS = 160

_mesh = plsc.VectorSubcoreMesh(
    core_axis_name="c", subcore_axis_name="s", num_cores=NC, num_subcores=NS
)


def _zero_rows(buf, nrows, ncols16):
    z = jnp.zeros((16,), jnp.float32)

    def body(i, carry):
        for k in range(ncols16):
            buf[i, pl.ds(k * 16, 16)] = z
        return carry

    lax.fori_loop(0, nrows, body, 0, unroll=4)


def _make_prop(F, GK, EBLK):
    BLK_PER_TILE = EDGES_PER_TILE // EBLK

    @functools.partial(
        pl.kernel,
        out_type=jax.ShapeDtypeStruct((NC, NP, F), jnp.float32),
        mesh=_mesh,
        compiler_params=pltpu.CompilerParams(use_tc_tiling_on_sc=False),
        scratch_types=[
            pltpu.VMEM((EDGES_PER_TILE,), jnp.int32),
            pltpu.VMEM((EDGES_PER_TILE,), jnp.int32),
            pltpu.VMEM((2 * GK * EBLK, F), jnp.float32),
            pltpu.VMEM((ZROWS, F), jnp.float32),
            pltpu.VMEM_SHARED((NP, F), jnp.float32),
            pltpu.SemaphoreType.DMA,
            pltpu.SemaphoreType.DMA,
        ],
    )
    def prop(h_hbm, ei_hbm, out_hbm, src_v, dst_v, rows_v, zbuf, acc, gsem, ssem):
        cid = lax.axis_index("c")
        sid = lax.axis_index("s")
        wid = sid * NC + cid

        _zero_rows(zbuf, ZROWS, F // 16)
        row0 = sid * ROWS_PER_TILE
        off = 0
        while off < ROWS_PER_TILE:
            n = min(ZROWS, ROWS_PER_TILE - off)
            pltpu.sync_copy(zbuf.at[pl.ds(0, n)], acc.at[pl.ds(row0 + off, n)])
            off += n
        plsc.subcore_barrier()

        base = wid * EDGES_PER_TILE
        pltpu.sync_copy(ei_hbm.at[0, pl.ds(base, EDGES_PER_TILE)], src_v)
        pltpu.sync_copy(ei_hbm.at[1, pl.ds(base, EDGES_PER_TILE)], dst_v)

        G = BLK_PER_TILE // GK

        def buf(g, b):
            return rows_v.at[pl.ds((g * GK + b) * EBLK, EBLK)]

        def idx(v, j):
            return v.at[pl.ds(j * EBLK, EBLK)]

        def fire_gathers(s, g):
            for b in range(GK):
                j = jnp.minimum(s * GK + b, BLK_PER_TILE - 1)
                pltpu.async_copy(h_hbm.at[idx(src_v, j)], buf(g, b), gsem)

        def drain_gathers(s, g):
            for b in range(GK):
                j = s * GK + b
                pltpu.make_async_copy(h_hbm.at[idx(src_v, j)], buf(g, b), gsem).wait()

        def fire_scatters(s, g):
            for b in range(GK):
                j = s * GK + b
                pltpu.async_copy(buf(g, b), acc.at[idx(dst_v, j)], ssem, add=True)

        def drain_scatters(s, g):
            for b in range(GK):
                j = s * GK + b
                pltpu.make_async_copy(buf(g, b), acc.at[idx(dst_v, j)], ssem).wait()

        fire_gathers(jnp.int32(0), jnp.int32(0))

        def step(s, carry):
            g = lax.rem(s, 2)

            drain_gathers(s, g)

            @pl.when(s > 0)
            def _():
                drain_scatters(s - 1, 1 - g)

            @pl.when(s < G - 1)
            def _():
                fire_gathers(s + 1, 1 - g)

            fire_scatters(s, g)
            return carry

        lax.fori_loop(0, G, step, 0)
        drain_scatters(jnp.int32(G - 1), jnp.int32((G - 1) % 2))
        plsc.subcore_barrier()

        pltpu.sync_copy(
            acc.at[pl.ds(row0, ROWS_PER_TILE)],
            out_hbm.at[cid, pl.ds(row0, ROWS_PER_TILE)],
        )

    return prop


@functools.partial(
    pl.kernel,
    out_type=jax.ShapeDtypeStruct((NC, NP, 16), jnp.float32),
    mesh=_mesh,
    compiler_params=pltpu.CompilerParams(use_tc_tiling_on_sc=False),
    scratch_types=[
        pltpu.VMEM((EDGES_PER_TILE,), jnp.int32),
        pltpu.VMEM((EBLK, 16), jnp.float32),
        pltpu.VMEM((ROWS_PER_TILE, 16), jnp.float32),
        pltpu.VMEM_SHARED((NP, 16), jnp.float32),
    ],
)
def _deg_kernel(ei_hbm, out_hbm, dst_v, ones_v, zbuf, acc):
    cid = lax.axis_index("c")
    sid = lax.axis_index("s")
    wid = sid * NC + cid

    one = jnp.ones((16,), jnp.float32)

    def fill(i, carry):
        ones_v[i, pl.ds(0, 16)] = one
        return carry

    lax.fori_loop(0, EBLK, fill, 0, unroll=4)

    _zero_rows(zbuf, ROWS_PER_TILE, 1)
    row0 = sid * ROWS_PER_TILE
    pltpu.sync_copy(zbuf, acc.at[pl.ds(row0, ROWS_PER_TILE)])
    plsc.subcore_barrier()

    pltpu.sync_copy(ei_hbm.at[1, pl.ds(wid * EDGES_PER_TILE, EDGES_PER_TILE)], dst_v)

    def step(j, carry):
        pltpu.sync_copy(ones_v, acc.at[dst_v.at[pl.ds(j * EBLK, EBLK)]], add=True)
        return carry

    lax.fori_loop(0, BLK_PER_TILE, step, 0)
    plsc.subcore_barrier()

    pltpu.sync_copy(
        acc.at[pl.ds(row0, ROWS_PER_TILE)],
        out_hbm.at[cid, pl.ds(row0, ROWS_PER_TILE)],
    )


_prop16 = _make_prop(F1, 2, 1000)
_prop32 = _make_prop(F2, 1, 1000)
_prop48 = _make_prop(F3P, 1, 400)


def _tc0_body(x_ref, w_ref, h_ref):
    h = lax.dot_general(
        x_ref[...], w_ref[...], (((1,), (0,)), ((), ())),
        preferred_element_type=jnp.float32,
    )
    h_ref[...] = jnp.pad(h, ((0, NP - N), (0, 0)))


def _tc1_body(h1_ref, degp_ref, h_ref, dinv_ref):
    deg = degp_ref[0, :, 0] + degp_ref[1, :, 0] + 1.0
    dinv = lax.rsqrt(deg)
    dinv_ref[...] = dinv
    h_ref[...] = h1_ref[...] * dinv[:, None]


def _tc_mid_body(sp_ref, hs_ref, dinv_ref, b_ref, w_ref, out_ref):
    dinv = dinv_ref[...]
    s = sp_ref[0] + sp_ref[1] + hs_ref[...]
    h = jnp.maximum(dinv[:, None] * s + b_ref[...], 0.0)
    h = lax.dot_general(
        h, w_ref[...], (((1,), (0,)), ((), ())),
        preferred_element_type=jnp.float32,
    )
    out_ref[...] = h * dinv[:, None]


def _tc4_body(sp_ref, hs_ref, dinv_ref, b_ref, out_ref):
    dinv = dinv_ref[...]
    h = dinv[:, None] * (sp_ref[0] + sp_ref[1] + hs_ref[...]) + b_ref[...]
    col = lax.broadcasted_iota(jnp.int32, h.shape, 1)
    h = jnp.where(col < C_OUT, h, -1e30)
    m = jnp.max(h, axis=1, keepdims=True)
    e = jnp.exp(h - m)
    ls = (h - m) - jnp.log(jnp.sum(e, axis=1, keepdims=True))
    out_ref[...] = ls[:N, :C_OUT]


def _tc0(x, W1):
    return pl.pallas_call(
        _tc0_body,
        out_shape=jax.ShapeDtypeStruct((NP, F1), jnp.float32),
    )(x, W1)


def _tc1(h1, degp):
    return pl.pallas_call(
        _tc1_body,
        out_shape=[
            jax.ShapeDtypeStruct((NP, F1), jnp.float32),
            jax.ShapeDtypeStruct((NP,), jnp.float32),
        ],
    )(h1, degp)


def _tc_mid(sp, hs, dinv, b2d, W, fout):
    return pl.pallas_call(
        _tc_mid_body,
        out_shape=jax.ShapeDtypeStruct((NP, fout), jnp.float32),
    )(sp, hs, dinv, b2d, W)


def _tc4(sp, hs, dinv, b2d):
    return pl.pallas_call(
        _tc4_body,
        out_shape=jax.ShapeDtypeStruct((N, C_OUT), jnp.float32),
    )(sp, hs, dinv, b2d)


def kernel(x, edge_index, W1, b1, W2, b2, W3, b3):
    W3p = jnp.pad(W3, ((0, 0), (0, F3P - C_OUT)))
    b3p = jnp.pad(b3, (0, F3P - C_OUT)).reshape(1, F3P)

    degp = _deg_kernel(edge_index)
    h1 = _tc0(x, W1)
    h1s, dinv = _tc1(h1, degp)
    s1 = _prop16(h1s, edge_index)
    h2s = _tc_mid(s1, h1s, dinv, b1.reshape(1, F1), W2, F2)
    s2 = _prop32(h2s, edge_index)
    h3s = _tc_mid(s2, h2s, dinv, b2.reshape(1, F2), W3p, F3P)
    s3 = _prop48(h3s, edge_index)
    return _tc4(s3, h3s, dinv, b3p)
